# baseline (speedup 1.0000x reference)
import functools

import jax
import jax.numpy as jnp
from jax import lax
from jax.experimental import pallas as pl
from jax.experimental.pallas import tpu as pltpu
from jax.experimental.pallas import tpu_sc as plsc

_N = 65536
_NC = 2
_NS = 16
_NW = _NC * _NS
_C = 4096
_G = _C // 16
_NSL = _N // _NS
_NRP = 4
_NCV = 2


@functools.lru_cache(maxsize=None)
def _spmv(nnz):
    full = nnz // _C
    tail = nnz - full * _C
    nchunks = full + (1 if tail else 0)
    steps = -(-nchunks // _NW)
    steps = -(-steps // _NRP) * _NRP

    mesh = plsc.VectorSubcoreMesh(core_axis_name="c", subcore_axis_name="s")

    @functools.partial(
        pl.kernel,
        out_type=jax.ShapeDtypeStruct((_NC, _N), jnp.float32),
        mesh=mesh,
        compiler_params=pltpu.CompilerParams(needs_layout_passes=False),
        scratch_types=[
            pltpu.VMEM((_N,), jnp.float32),
            *[pltpu.VMEM((_C,), jnp.int32) for _ in range(_NRP)],
            *[pltpu.VMEM((_C,), jnp.int32) for _ in range(_NCV)],
            *[pltpu.VMEM((_C,), jnp.float32) for _ in range(_NCV)],
            *[pltpu.VMEM((_C,), jnp.float32) for _ in range(_NRP)],
            pltpu.VMEM_SHARED((_N,), jnp.float32),
            pltpu.VMEM_SHARED((_N,), jnp.float32),
            *[pltpu.SemaphoreType.DMA for _ in range(_NCV + _NRP)],
        ],
    )
    def k(rows_hbm, cols_hbm, vals_hbm, rows_t, cols_t, vals_t,
          x_hbm, out_hbm,
          x_l, rb0, rb1, rb2, rb3, cb0, cb1, vb0, vb1, pb0, pb1, pb2, pb3,
          y_sh, x_sh, is0, is1, ss0, ss1, ss2, ss3):
        rows_b = (rb0, rb1, rb2, rb3)
        cols_b = (cb0, cb1)
        vals_b = (vb0, vb1)
        prod_b = (pb0, pb1, pb2, pb3)
        in_sems = (is0, is1)
        sc_sems = (ss0, ss1, ss2, ss3)
        c = lax.axis_index("c")
        s = lax.axis_index("s")
        w = c * _NS + s

        zsl = pl.ds(s * _NSL, _NSL)
        pltpu.sync_copy(x_hbm.at[zsl], x_sh.at[zsl])
        z16 = jnp.zeros((16,), jnp.float32)

        @plsc.parallel_loop(0, _G, 1, unroll=8)
        def _z(g):
            pb0[pl.ds(g * 16, 16)] = z16

        pltpu.sync_copy(pb0, y_sh.at[zsl])
        plsc.subcore_barrier()
        pltpu.sync_copy(x_sh, x_l)

        def fire_inputs(t, rp, cv):
            cid = t * _NW + w
            base = cid * _C

            @pl.when(cid < full)
            def _():
                pltpu.async_copy(rows_hbm.at[pl.ds(base, _C)],
                                 rows_b[rp], in_sems[cv])
                pltpu.async_copy(cols_hbm.at[pl.ds(base, _C)],
                                 cols_b[cv], in_sems[cv])
                pltpu.async_copy(vals_hbm.at[pl.ds(base, _C)],
                                 vals_b[cv], in_sems[cv])

            if tail:
                @pl.when(cid == full)
                def _():
                    pltpu.async_copy(rows_t, rows_b[rp], in_sems[cv])
                    pltpu.async_copy(cols_t, cols_b[cv], in_sems[cv])
                    pltpu.async_copy(vals_t, vals_b[cv], in_sems[cv])

        def wait_inputs(t, rp, cv):
            cid = t * _NW + w

            @pl.when(cid <= nchunks - 1)
            def _():
                pltpu.make_async_copy(rows_hbm.at[pl.ds(0, _C)],
                                      rows_b[rp], in_sems[cv]).wait()
                pltpu.make_async_copy(cols_hbm.at[pl.ds(0, _C)],
                                      cols_b[cv], in_sems[cv]).wait()
                pltpu.make_async_copy(vals_hbm.at[pl.ds(0, _C)],
                                      vals_b[cv], in_sems[cv]).wait()

        def compute(rp, cv):
            @plsc.parallel_loop(0, _G, 1, unroll=8)
            def g_body(g):
                sl = pl.ds(g * 16, 16)
                idx = cols_b[cv][sl]
                xv = plsc.load_gather(x_l, [idx])
                prod_b[rp][sl] = xv * vals_b[cv][sl]

        def fire_scatter(rp):
            pltpu.async_copy(prod_b[rp], y_sh.at[rows_b[rp]],
                             sc_sems[rp], add=True)

        def wait_scatter(rp):
            pltpu.make_async_copy(prod_b[rp], y_sh.at[rows_b[rp]],
                                  sc_sems[rp]).wait()

        fire_inputs(0, 0, 0)

        def pipe_body(i, carry):
            for j in range(_NRP):
                t = i * _NRP + j
                rp = j
                cv = j % _NCV
                nrp = (j + 1) % _NRP
                ncv = (j + 1) % _NCV
                tp = t - (_NRP - 1)
                cidp = tp * _NW + w

                @pl.when((tp >= 0) & (cidp <= nchunks - 1))
                def _():
                    wait_scatter(nrp)

                fire_inputs(t + 1, nrp, ncv)
                wait_inputs(t, rp, cv)
                cid = t * _NW + w

                @pl.when(cid <= nchunks - 1)
                def _():
                    compute(rp, cv)
                    fire_scatter(rp)
            return carry

        lax.fori_loop(0, steps // _NRP, pipe_body, 0)

        for t in range(steps - (_NRP - 1), steps):
            cid = t * _NW + w

            @pl.when((t >= 0) & (cid <= nchunks - 1))
            def _():
                wait_scatter(t % _NRP)

        plsc.subcore_barrier()
        pltpu.sync_copy(y_sh.at[zsl], out_hbm.at[c, zsl])

    return k


def _combine(partials):
    def body(p_ref, o_ref):
        o_ref[...] = p_ref[0] + p_ref[1]

    return pl.pallas_call(
        body,
        out_shape=jax.ShapeDtypeStruct((_N,), jnp.float32),
    )(partials)


def kernel(rows, cols, vals, x):
    nnz = rows.shape[0]
    full = nnz // _C
    tail = nnz - full * _C
    pad = _C - tail if tail else 0
    rows_t = jnp.pad(rows[full * _C:], (0, pad))
    cols_t = jnp.pad(cols[full * _C:], (0, pad))
    vals_t = jnp.pad(vals[full * _C:], (0, pad))
    partials = _spmv(nnz)(rows, cols, vals, rows_t, cols_t, vals_t, x)
    y = _combine(partials)
    return y.astype(jnp.float64)


# speedup vs baseline: 1.0073x; 1.0073x over previous
import functools

import jax
import jax.numpy as jnp
from jax import lax
from jax.experimental import pallas as pl
from jax.experimental.pallas import tpu as pltpu
from jax.experimental.pallas import tpu_sc as plsc

_N = 65536
_NC = 2
_NS = 16
_NW = _NC * _NS
_C = 4096
_G = _C // 16
_NSL = _N // _NS
_NRP = 4
_NCV = 2


@functools.lru_cache(maxsize=None)
def _spmv(nnz):
    full = nnz // _C
    tail = nnz - full * _C
    nchunks = full + (1 if tail else 0)
    steps = -(-nchunks // _NW)
    steps = -(-steps // _NRP) * _NRP

    mesh = plsc.VectorSubcoreMesh(core_axis_name="c", subcore_axis_name="s")

    @functools.partial(
        pl.kernel,
        out_type=jax.ShapeDtypeStruct((_NC, _N), jnp.float32),
        mesh=mesh,
        compiler_params=pltpu.CompilerParams(needs_layout_passes=False),
        scratch_types=[
            pltpu.VMEM((_N,), jnp.float32),
            *[pltpu.VMEM((_C,), jnp.int32) for _ in range(_NRP)],
            *[pltpu.VMEM((_C,), jnp.int32) for _ in range(_NCV)],
            *[pltpu.VMEM((_C,), jnp.float32) for _ in range(_NCV)],
            *[pltpu.VMEM((_C,), jnp.float32) for _ in range(_NRP)],
            pltpu.VMEM_SHARED((_N,), jnp.float32),
            pltpu.VMEM_SHARED((_N,), jnp.float32),
            *[pltpu.SemaphoreType.DMA for _ in range(_NCV + _NRP)],
        ],
    )
    def k(rows_hbm, cols_hbm, vals_hbm, rc_t, vals_t,
          x_hbm, out_hbm,
          x_l, rb0, rb1, rb2, rb3, cb0, cb1, vb0, vb1, pb0, pb1, pb2, pb3,
          y_sh, x_sh, is0, is1, ss0, ss1, ss2, ss3):
        rows_b = (rb0, rb1, rb2, rb3)
        cols_b = (cb0, cb1)
        vals_b = (vb0, vb1)
        prod_b = (pb0, pb1, pb2, pb3)
        in_sems = (is0, is1)
        sc_sems = (ss0, ss1, ss2, ss3)
        c = lax.axis_index("c")
        s = lax.axis_index("s")
        w = c * _NS + s

        zsl = pl.ds(s * _NSL, _NSL)
        pltpu.sync_copy(x_hbm.at[zsl], x_sh.at[zsl])
        z16 = jnp.zeros((16,), jnp.float32)

        @plsc.parallel_loop(0, _G, 1, unroll=8)
        def _z(g):
            pb0[pl.ds(g * 16, 16)] = z16

        pltpu.sync_copy(pb0, y_sh.at[zsl])
        plsc.subcore_barrier()
        pltpu.sync_copy(x_sh, x_l)

        def fire_inputs(t, rp, cv):
            cid = t * _NW + w
            base = cid * _C

            @pl.when(cid < full)
            def _():
                pltpu.async_copy(rows_hbm.at[pl.ds(base, _C)],
                                 rows_b[rp], in_sems[cv])
                pltpu.async_copy(cols_hbm.at[pl.ds(base, _C)],
                                 cols_b[cv], in_sems[cv])
                pltpu.async_copy(vals_hbm.at[pl.ds(base, _C)],
                                 vals_b[cv], in_sems[cv])

            if tail:
                @pl.when(cid == full)
                def _():
                    pltpu.async_copy(rc_t.at[pl.ds(0, _C)],
                                     rows_b[rp], in_sems[cv])
                    pltpu.async_copy(rc_t.at[pl.ds(_C, _C)],
                                     cols_b[cv], in_sems[cv])
                    pltpu.async_copy(vals_t, vals_b[cv], in_sems[cv])

        def wait_inputs(t, rp, cv):
            cid = t * _NW + w

            @pl.when(cid <= nchunks - 1)
            def _():
                pltpu.make_async_copy(rows_hbm.at[pl.ds(0, _C)],
                                      rows_b[rp], in_sems[cv]).wait()
                pltpu.make_async_copy(cols_hbm.at[pl.ds(0, _C)],
                                      cols_b[cv], in_sems[cv]).wait()
                pltpu.make_async_copy(vals_hbm.at[pl.ds(0, _C)],
                                      vals_b[cv], in_sems[cv]).wait()

        def compute(rp, cv):
            @plsc.parallel_loop(0, _G, 1, unroll=8)
            def g_body(g):
                sl = pl.ds(g * 16, 16)
                idx = cols_b[cv][sl]
                xv = plsc.load_gather(x_l, [idx])
                prod_b[rp][sl] = xv * vals_b[cv][sl]

        def fire_scatter(rp):
            pltpu.async_copy(prod_b[rp], y_sh.at[rows_b[rp]],
                             sc_sems[rp], add=True)

        def wait_scatter(rp):
            pltpu.make_async_copy(prod_b[rp], y_sh.at[rows_b[rp]],
                                  sc_sems[rp]).wait()

        fire_inputs(0, 0, 0)

        def pipe_body(i, carry):
            for j in range(_NRP):
                t = i * _NRP + j
                rp = j
                cv = j % _NCV
                nrp = (j + 1) % _NRP
                ncv = (j + 1) % _NCV
                tp = t - (_NRP - 1)
                cidp = tp * _NW + w

                @pl.when((tp >= 0) & (cidp <= nchunks - 1))
                def _():
                    wait_scatter(nrp)

                fire_inputs(t + 1, nrp, ncv)
                wait_inputs(t, rp, cv)
                cid = t * _NW + w

                @pl.when(cid <= nchunks - 1)
                def _():
                    compute(rp, cv)
                    fire_scatter(rp)
            return carry

        lax.fori_loop(0, steps // _NRP, pipe_body, 0)

        for t in range(steps - (_NRP - 1), steps):
            cid = t * _NW + w

            @pl.when((t >= 0) & (cid <= nchunks - 1))
            def _():
                wait_scatter(t % _NRP)

        plsc.subcore_barrier()
        pltpu.sync_copy(y_sh.at[zsl], out_hbm.at[c, zsl])

    return k


def _combine(partials):
    def body(p_ref, o_ref):
        o_ref[...] = p_ref[0] + p_ref[1]

    return pl.pallas_call(
        body,
        out_shape=jax.ShapeDtypeStruct((_N,), jnp.float32),
    )(partials)


def kernel(rows, cols, vals, x):
    nnz = rows.shape[0]
    full = nnz // _C
    tail = nnz - full * _C
    pad = _C - tail if tail else 0
    rc_t = (jnp.zeros((2 * _C,), jnp.int32)
            .at[:tail].set(rows[full * _C:])
            .at[_C:_C + tail].set(cols[full * _C:]))
    vals_t = jnp.pad(vals[full * _C:], (0, pad))
    partials = _spmv(nnz)(rows, cols, vals, rc_t, vals_t, x)
    y = _combine(partials)
    return y.astype(jnp.float64)
